# baseline (speedup 1.0000x reference)
import functools

import jax
import jax.numpy as jnp
from jax import lax
from jax.experimental import pallas as pl
from jax.experimental.pallas import tpu as pltpu
from jax.experimental.pallas import tpu_sc as plsc

N_ATOM = 10000
N_BOND = 320000
D_BOND = 128
D_GLOBAL = 64
NC = 2
NS = 16
NW = NC * NS
E_PER_TILE = N_BOND // NW
CHUNK = 80
N_CHUNKS = E_PER_TILE // CHUNK
A_PAD = 10240
A_PER_TILE = A_PAD // NS
ZROWS = 40

CQ = 128
CR = 80
CE = 2048
CN = -(-N_BOND // CE)

_sc_mesh = plsc.VectorSubcoreMesh(
    core_axis_name="c", subcore_axis_name="s", num_cores=NC, num_subcores=NS)


@functools.partial(
    pl.kernel,
    out_type=jax.ShapeDtypeStruct((NC, A_PAD, D_BOND), jnp.float32),
    mesh=_sc_mesh,
    scratch_types=(
        pltpu.VMEM((CHUNK,), jnp.int32),
        pltpu.VMEM((CHUNK,), jnp.int32),
        pltpu.VMEM((CHUNK,), jnp.int32),
        pltpu.VMEM((CHUNK,), jnp.int32),
        pltpu.VMEM((CHUNK,), jnp.int32),
        pltpu.VMEM((CHUNK,), jnp.int32),
        pltpu.VMEM((CHUNK,), jnp.int32),
        pltpu.VMEM((CHUNK,), jnp.int32),
        pltpu.VMEM((CHUNK,), jnp.int32),
        pltpu.VMEM((CHUNK, D_BOND), jnp.float32),
        pltpu.VMEM((CHUNK, D_BOND), jnp.float32),
        pltpu.VMEM((CHUNK, D_BOND), jnp.float32),
        pltpu.VMEM((ZROWS, D_BOND), jnp.float32),
        pltpu.VMEM_SHARED((A_PAD, D_BOND), jnp.float32),
        pltpu.SemaphoreType.DMA,
        pltpu.SemaphoreType.DMA,
        pltpu.SemaphoreType.DMA,
        pltpu.SemaphoreType.DMA,
        pltpu.SemaphoreType.DMA,
        pltpu.SemaphoreType.DMA,
        pltpu.SemaphoreType.DMA,
        pltpu.SemaphoreType.DMA,
        pltpu.SemaphoreType.DMA,
    ),
)
def _sc_segment_sum(bond_hbm, src_hbm, dst_hbm, sum_out,
                    si0, si1, si2, di0, di1, di2, sd0, sd1, sd2,
                    rows0, rows1, rows2, zrow, acc_sp,
                    smi0, smi1, smi2, smg0, smg1, smg2, sms0, sms1, sms2):
    c = lax.axis_index("c")
    s = lax.axis_index("s")
    w = s * NC + c
    tile_base = w * E_PER_TILE

    si = (si0, si1, si2)
    di = (di0, di1, di2)
    sd = (sd0, sd1, sd2)
    rows = (rows0, rows1, rows2)
    smi = (smi0, smi1, smi2)
    smg = (smg0, smg1, smg2)
    sms = (sms0, sms1, sms2)

    z16 = jnp.zeros((16,), jnp.float32)

    def zero_feat(i, carry):
        zrow[i // 8, pl.ds((i % 8) * 16, 16)] = z16
        return carry
    lax.fori_loop(0, ZROWS * 8, zero_feat, 0)

    def zero_stripe(j, carry):
        pltpu.sync_copy(zrow, acc_sp.at[pl.ds(s * A_PER_TILE + j * ZROWS, ZROWS)])
        return carry
    lax.fori_loop(0, A_PER_TILE // ZROWS, zero_stripe, 0)

    plsc.subcore_barrier()

    def idx_start(j, b):
        base = tile_base + j * CHUNK
        pltpu.async_copy(src_hbm.at[pl.ds(base, CHUNK)], si[b], smi[b])
        pltpu.async_copy(dst_hbm.at[pl.ds(base, CHUNK)], di[b], smi[b])

    def idx_wait(j, b):
        base = tile_base + j * CHUNK
        pltpu.make_async_copy(src_hbm.at[pl.ds(base, CHUNK)], si[b], smi[b]).wait()
        pltpu.make_async_copy(dst_hbm.at[pl.ds(base, CHUNK)], di[b], smi[b]).wait()

    def gather_start(b):
        pltpu.async_copy(bond_hbm.at[si[b]], rows[b], smg[b])

    def gather_wait(b):
        pltpu.make_async_copy(bond_hbm.at[si[b]], rows[b], smg[b]).wait()

    def scatter_start(b):
        for k in range(CHUNK // 16):
            sd[b][pl.ds(k * 16, 16)] = di[b][pl.ds(k * 16, 16)]
        pltpu.async_copy(rows[b], acc_sp.at[sd[b]], sms[b], add=True)

    def scatter_wait(b):
        pltpu.make_async_copy(rows[b], acc_sp.at[sd[b]], sms[b]).wait()

    idx_start(0, 0)
    idx_wait(0, 0)
    idx_start(1, 1)
    gather_start(0)
    idx_wait(1, 1)
    idx_start(2, 2)
    gather_start(1)
    gather_wait(0)
    scatter_start(0)

    def section(j, b, bp1, bp2):
        idx_wait(j + 1, bp1)

        @pl.when(j + 2 < N_CHUNKS)
        def _():
            idx_start(j + 2, bp2)

        @pl.when(j + 1 < N_CHUNKS)
        def _():
            gather_start(bp1)
        gather_wait(b)
        scatter_wait(bp2)
        scatter_start(b)

    def triple(k, carry):
        section(3 * k + 1, 1, 2, 0)
        section(3 * k + 2, 2, 0, 1)
        section(3 * k + 3, 0, 1, 2)
        return carry
    lax.fori_loop(0, (N_CHUNKS - 2) // 3, triple, 0)

    gather_wait(1)
    scatter_wait(0)
    scatter_start(1)
    scatter_wait(1)

    plsc.subcore_barrier()

    pltpu.sync_copy(acc_sp.at[pl.ds(s * A_PER_TILE, A_PER_TILE)],
                    sum_out.at[c, pl.ds(s * A_PER_TILE, A_PER_TILE)])


def _count_body(dst_ref, out_ref):
    d = dst_ref[0, 0, :]
    q = d // CR
    r = d - q * CR
    ohq = (q[:, None] == lax.broadcasted_iota(jnp.int32, (CE, CQ), 1)
           ).astype(jnp.bfloat16)
    ohr = (r[:, None] == lax.broadcasted_iota(jnp.int32, (CE, CR), 1)
           ).astype(jnp.bfloat16)
    part = lax.dot_general(ohq, ohr, (((0,), (0,)), ((), ())),
                           preferred_element_type=jnp.float32)

    @pl.when(pl.program_id(0) == 0)
    def _():
        out_ref[...] = jnp.zeros_like(out_ref)

    out_ref[...] += part


def _tc_counts(dst3):
    return pl.pallas_call(
        _count_body,
        grid=(CN,),
        in_specs=[pl.BlockSpec((1, 1, CE), lambda i: (i, 0, 0))],
        out_specs=pl.BlockSpec((CQ, CR), lambda i: (0, 0)),
        out_shape=jax.ShapeDtypeStruct((CQ, CR), jnp.float32),
    )(dst3)


def _softplus(x):
    return jnp.maximum(x, 0.0) + jnp.log1p(jnp.exp(-jnp.abs(x)))


def _mlp_body(master_ref, sums_ref, cnts_ref, g2a_ref, glob_ref,
              w1_ref, b1_ref, w2_ref, b2_ref, w3_ref, b3_ref, out_ref):
    f32 = jnp.float32
    blk = master_ref.shape[0]
    cnt = cnts_ref[...]
    mean_b = (sums_ref[0] + sums_ref[1]) / jnp.maximum(cnt, 1.0)
    g = g2a_ref[:, 0]
    onehot = (g[:, None] == lax.broadcasted_iota(jnp.int32, (blk, D_GLOBAL), 1)
              ).astype(f32)
    mean_g = jnp.dot(onehot, glob_ref[...], preferred_element_type=f32)
    h = jnp.dot(master_ref[...], w1_ref[0:128, :], preferred_element_type=f32)
    h += jnp.dot(mean_b, w1_ref[128:256, :], preferred_element_type=f32)
    h += jnp.dot(mean_g, w1_ref[256:320, :], preferred_element_type=f32)
    h = _softplus(h + b1_ref[...])
    h = _softplus(jnp.dot(h, w2_ref[...], preferred_element_type=f32) + b2_ref[...])
    out_ref[...] = jnp.dot(h, w3_ref[...], preferred_element_type=f32) + b3_ref[...]


def _tc_mlp(master, sums, cnts, g2a2d, globalf, W1, b1, W2, b2, W3, b3):
    B = 2000
    grid = (N_ATOM // B,)
    fixed = lambda i: (0, 0)
    return pl.pallas_call(
        _mlp_body,
        grid=grid,
        in_specs=[
            pl.BlockSpec((B, 128), lambda i: (i, 0)),
            pl.BlockSpec((NC, B, 128), lambda i: (0, i, 0)),
            pl.BlockSpec((B, 1), lambda i: (i, 0)),
            pl.BlockSpec((B, 1), lambda i: (i, 0)),
            pl.BlockSpec((D_GLOBAL, D_GLOBAL), fixed),
            pl.BlockSpec((320, 256), fixed),
            pl.BlockSpec((1, 256), fixed),
            pl.BlockSpec((256, 256), fixed),
            pl.BlockSpec((1, 256), fixed),
            pl.BlockSpec((256, 128), fixed),
            pl.BlockSpec((1, 128), fixed),
        ],
        out_specs=pl.BlockSpec((B, 128), lambda i: (i, 0)),
        out_shape=jax.ShapeDtypeStruct((N_ATOM, 128), jnp.float32),
    )(master, sums, cnts, g2a2d, globalf,
      W1, b1.reshape(1, -1), W2, b2.reshape(1, -1), W3, b3.reshape(1, -1))


def kernel(master_feats, bond_feats, global_feats, b2a_src, b2a_dst,
           g2a_src, g2a_dst, W1, b1, W2, b2, W3, b3):
    del g2a_dst
    b2a_src = b2a_src.astype(jnp.int32)
    b2a_dst = b2a_dst.astype(jnp.int32)
    sums = _sc_segment_sum(bond_feats, b2a_src, b2a_dst)
    dst_pad = jnp.concatenate(
        [b2a_dst, jnp.full((CN * CE - N_BOND,), A_PAD, jnp.int32)])
    cnt_qr = _tc_counts(dst_pad.reshape(CN, 1, CE))
    cnts = cnt_qr.reshape(CQ * CR, 1)[:N_ATOM]
    g2a2d = g2a_src.astype(jnp.int32).reshape(N_ATOM, 1)
    return _tc_mlp(master_feats, sums, cnts, g2a2d, global_feats,
                   W1, b1, W2, b2, W3, b3)


# speedup vs baseline: 1.0180x; 1.0180x over previous
import functools

import jax
import jax.numpy as jnp
from jax import lax
from jax.experimental import pallas as pl
from jax.experimental.pallas import tpu as pltpu
from jax.experimental.pallas import tpu_sc as plsc

N_ATOM = 10000
N_BOND = 320000
D_BOND = 128
D_GLOBAL = 64
NC = 2
NS = 16
NW = NC * NS
E_PER_TILE = N_BOND // NW
CHUNK = 80
N_CHUNKS = E_PER_TILE // CHUNK
A_PAD = 10240
A_PER_TILE = A_PAD // NS
ZROWS = 40

CQ = 128
CR = 80
CE = 2048
CN = -(-N_BOND // CE)

_sc_mesh = plsc.VectorSubcoreMesh(
    core_axis_name="c", subcore_axis_name="s", num_cores=NC, num_subcores=NS)


@functools.partial(
    pl.kernel,
    out_type=jax.ShapeDtypeStruct((NC, A_PAD, D_BOND), jnp.float32),
    mesh=_sc_mesh,
    scratch_types=(
        pltpu.VMEM((CHUNK,), jnp.int32),
        pltpu.VMEM((CHUNK,), jnp.int32),
        pltpu.VMEM((CHUNK,), jnp.int32),
        pltpu.VMEM((CHUNK,), jnp.int32),
        pltpu.VMEM((CHUNK,), jnp.int32),
        pltpu.VMEM((CHUNK,), jnp.int32),
        pltpu.VMEM((CHUNK,), jnp.int32),
        pltpu.VMEM((CHUNK,), jnp.int32),
        pltpu.VMEM((CHUNK,), jnp.int32),
        pltpu.VMEM((CHUNK, D_BOND), jnp.float32),
        pltpu.VMEM((CHUNK, D_BOND), jnp.float32),
        pltpu.VMEM((CHUNK, D_BOND), jnp.float32),
        pltpu.VMEM((ZROWS, D_BOND), jnp.float32),
        pltpu.VMEM_SHARED((A_PAD, D_BOND), jnp.float32),
        pltpu.SemaphoreType.DMA,
        pltpu.SemaphoreType.DMA,
        pltpu.SemaphoreType.DMA,
        pltpu.SemaphoreType.DMA,
        pltpu.SemaphoreType.DMA,
        pltpu.SemaphoreType.DMA,
        pltpu.SemaphoreType.DMA,
        pltpu.SemaphoreType.DMA,
        pltpu.SemaphoreType.DMA,
    ),
)
def _sc_segment_sum(bond_hbm, src_hbm, dst_hbm, sum_out,
                    si0, si1, si2, di0, di1, di2, sd0, sd1, sd2,
                    rows0, rows1, rows2, zrow, acc_sp,
                    smi0, smi1, smi2, smg0, smg1, smg2, sms0, sms1, sms2):
    c = lax.axis_index("c")
    s = lax.axis_index("s")
    w = s * NC + c
    tile_base = w * E_PER_TILE

    si = (si0, si1, si2)
    di = (di0, di1, di2)
    sd = (sd0, sd1, sd2)
    rows = (rows0, rows1, rows2)
    smi = (smi0, smi1, smi2)
    smg = (smg0, smg1, smg2)
    sms = (sms0, sms1, sms2)

    z16 = jnp.zeros((16,), jnp.float32)

    def zero_feat(i, carry):
        zrow[i // 8, pl.ds((i % 8) * 16, 16)] = z16
        return carry
    lax.fori_loop(0, ZROWS * 8, zero_feat, 0)

    def zero_stripe(j, carry):
        pltpu.sync_copy(zrow, acc_sp.at[pl.ds(s * A_PER_TILE + j * ZROWS, ZROWS)])
        return carry
    lax.fori_loop(0, A_PER_TILE // ZROWS, zero_stripe, 0)

    plsc.subcore_barrier()

    def idx_start(j, b):
        base = tile_base + j * CHUNK
        pltpu.async_copy(src_hbm.at[pl.ds(base, CHUNK)], si[b], smi[b])
        pltpu.async_copy(dst_hbm.at[pl.ds(base, CHUNK)], di[b], smi[b])

    def idx_wait(j, b):
        base = tile_base + j * CHUNK
        pltpu.make_async_copy(src_hbm.at[pl.ds(base, CHUNK)], si[b], smi[b]).wait()
        pltpu.make_async_copy(dst_hbm.at[pl.ds(base, CHUNK)], di[b], smi[b]).wait()

    def gather_start(b):
        pltpu.async_copy(bond_hbm.at[si[b]], rows[b], smg[b])

    def gather_wait(b):
        pltpu.make_async_copy(bond_hbm.at[si[b]], rows[b], smg[b]).wait()

    def scatter_start(b):
        for k in range(CHUNK // 16):
            sd[b][pl.ds(k * 16, 16)] = di[b][pl.ds(k * 16, 16)]
        pltpu.async_copy(rows[b], acc_sp.at[sd[b]], sms[b], add=True)

    def scatter_wait(b):
        pltpu.make_async_copy(rows[b], acc_sp.at[sd[b]], sms[b]).wait()

    idx_start(0, 0)
    idx_wait(0, 0)
    idx_start(1, 1)
    gather_start(0)
    idx_wait(1, 1)
    idx_start(2, 2)
    gather_start(1)
    gather_wait(0)
    scatter_start(0)

    def section(j, b, bp1, bp2):
        idx_wait(j + 1, bp1)

        @pl.when(j + 2 < N_CHUNKS)
        def _():
            idx_start(j + 2, bp2)

        @pl.when(j + 1 < N_CHUNKS)
        def _():
            gather_start(bp1)
        gather_wait(b)
        scatter_wait(bp2)
        scatter_start(b)

    def triple(k, carry):
        section(3 * k + 1, 1, 2, 0)
        section(3 * k + 2, 2, 0, 1)
        section(3 * k + 3, 0, 1, 2)
        return carry
    lax.fori_loop(0, (N_CHUNKS - 2) // 3, triple, 0)

    gather_wait(1)
    scatter_wait(0)
    scatter_start(1)
    scatter_wait(1)

    plsc.subcore_barrier()

    pltpu.sync_copy(acc_sp.at[pl.ds(s * A_PER_TILE, A_PER_TILE)],
                    sum_out.at[c, pl.ds(s * A_PER_TILE, A_PER_TILE)])


def _count_body(dst_ref, out_ref):
    d = dst_ref[0, 0, :]
    q = d // CR
    r = d - q * CR
    ohq = (q[:, None] == lax.broadcasted_iota(jnp.int32, (CE, CQ), 1)
           ).astype(jnp.float32)
    ohr = (r[:, None] == lax.broadcasted_iota(jnp.int32, (CE, CR), 1)
           ).astype(jnp.float32)
    part = lax.dot_general(ohq, ohr, (((0,), (0,)), ((), ())),
                           preferred_element_type=jnp.float32)

    @pl.when(pl.program_id(0) == 0)
    def _():
        out_ref[...] = jnp.zeros_like(out_ref)

    out_ref[...] += part


def _tc_counts(dst3):
    return pl.pallas_call(
        _count_body,
        grid=(CN,),
        in_specs=[pl.BlockSpec((1, 1, CE), lambda i: (i, 0, 0))],
        out_specs=pl.BlockSpec((CQ, CR), lambda i: (0, 0)),
        out_shape=jax.ShapeDtypeStruct((CQ, CR), jnp.float32),
    )(dst3)


def _softplus(x):
    return jnp.maximum(x, 0.0) + jnp.log1p(jnp.exp(-jnp.abs(x)))


def _mlp_body(master_ref, sums_ref, cnts_ref, g2a_ref, glob_ref,
              w1_ref, b1_ref, w2_ref, b2_ref, w3_ref, b3_ref, out_ref):
    f32 = jnp.float32
    blk = master_ref.shape[0]
    cnt = cnts_ref[...]
    mean_b = (sums_ref[0] + sums_ref[1]) / jnp.maximum(cnt, 1.0)
    g = g2a_ref[:, 0]
    onehot = (g[:, None] == lax.broadcasted_iota(jnp.int32, (blk, D_GLOBAL), 1)
              ).astype(f32)
    mean_g = jnp.dot(onehot, glob_ref[...], preferred_element_type=f32)
    h = jnp.dot(master_ref[...], w1_ref[0:128, :], preferred_element_type=f32)
    h += jnp.dot(mean_b, w1_ref[128:256, :], preferred_element_type=f32)
    h += jnp.dot(mean_g, w1_ref[256:320, :], preferred_element_type=f32)
    h = _softplus(h + b1_ref[...])
    h = _softplus(jnp.dot(h, w2_ref[...], preferred_element_type=f32) + b2_ref[...])
    out_ref[...] = jnp.dot(h, w3_ref[...], preferred_element_type=f32) + b3_ref[...]


def _tc_mlp(master, sums, cnts, g2a2d, globalf, W1, b1, W2, b2, W3, b3):
    B = 2000
    grid = (N_ATOM // B,)
    fixed = lambda i: (0, 0)
    return pl.pallas_call(
        _mlp_body,
        grid=grid,
        in_specs=[
            pl.BlockSpec((B, 128), lambda i: (i, 0)),
            pl.BlockSpec((NC, B, 128), lambda i: (0, i, 0)),
            pl.BlockSpec((B, 1), lambda i: (i, 0)),
            pl.BlockSpec((B, 1), lambda i: (i, 0)),
            pl.BlockSpec((D_GLOBAL, D_GLOBAL), fixed),
            pl.BlockSpec((320, 256), fixed),
            pl.BlockSpec((1, 256), fixed),
            pl.BlockSpec((256, 256), fixed),
            pl.BlockSpec((1, 256), fixed),
            pl.BlockSpec((256, 128), fixed),
            pl.BlockSpec((1, 128), fixed),
        ],
        out_specs=pl.BlockSpec((B, 128), lambda i: (i, 0)),
        out_shape=jax.ShapeDtypeStruct((N_ATOM, 128), jnp.float32),
    )(master, sums, cnts, g2a2d, globalf,
      W1, b1.reshape(1, -1), W2, b2.reshape(1, -1), W3, b3.reshape(1, -1))


def kernel(master_feats, bond_feats, global_feats, b2a_src, b2a_dst,
           g2a_src, g2a_dst, W1, b1, W2, b2, W3, b3):
    del g2a_dst
    b2a_src = b2a_src.astype(jnp.int32)
    b2a_dst = b2a_dst.astype(jnp.int32)
    sums = _sc_segment_sum(bond_feats, b2a_src, b2a_dst)
    dst_pad = jnp.concatenate(
        [b2a_dst, jnp.full((CN * CE - N_BOND,), A_PAD, jnp.int32)])
    cnt_qr = _tc_counts(dst_pad.reshape(CN, 1, CE))
    cnts = cnt_qr.reshape(CQ * CR, 1)[:N_ATOM]
    g2a2d = g2a_src.astype(jnp.int32).reshape(N_ATOM, 1)
    return _tc_mlp(master_feats, sums, cnts, g2a2d, global_feats,
                   W1, b1, W2, b2, W3, b3)
